# baseline (speedup 1.0000x reference)
import functools

import jax
import jax.numpy as jnp
from jax import lax
from jax.experimental import pallas as pl
from jax.experimental.pallas import tpu as pltpu
from jax.experimental.pallas import tpu_sc as plsc

_NC = 2
_NS = 16
_NW = _NC * _NS
_CH = 64
_F0 = 0.70


def _sc_mesh():
    return plsc.VectorSubcoreMesh(
        core_axis_name="c", subcore_axis_name="s", num_cores=_NC, num_subcores=_NS
    )


def _make_degree_kernel(ept, n_pad, interpret=False):
    @functools.partial(
        pl.kernel,
        out_type=jax.ShapeDtypeStruct((_NW, n_pad), jnp.float32),
        mesh=_sc_mesh(),
        scratch_types=[
            pltpu.VMEM((n_pad,), jnp.float32),
            pltpu.VMEM((ept,), jnp.int32),
        ],
        compiler_params=pltpu.CompilerParams(needs_layout_passes=False),
        interpret=interpret,
    )
    def deg_kernel(dst_hbm, out_hbm, deg_v, idx_v):
        wid = lax.axis_index("s") * _NC + lax.axis_index("c")

        def zero_body(i, _):
            deg_v[pl.ds(i * 16, 16)] = jnp.zeros((16,), jnp.float32)
            return 0

        lax.fori_loop(0, n_pad // 16, zero_body, 0)

        pltpu.sync_copy(dst_hbm.at[pl.ds(wid * ept, ept)], idx_v)

        ones = jnp.ones((16,), jnp.float32)

        def count_body(i, _):
            idx16 = idx_v[pl.ds(i * 16, 16)]
            plsc.addupdate_scatter(deg_v, [idx16], ones)
            return 0

        lax.fori_loop(0, ept // 16, count_body, 0)

        pltpu.sync_copy(deg_v, out_hbm.at[wid])

    return deg_kernel


def _make_edge_kernel(nch_pair, w, acc_rows, nbuf, frac0, interpret=False):
    nch0 = int(round(nch_pair * frac0))
    nch1 = nch_pair - nch0
    rpt = acc_rows // _NS
    k = nbuf - 1
    nidx = nbuf + 3

    @functools.partial(
        pl.kernel,
        out_type=jax.ShapeDtypeStruct((_NC, acc_rows, w), jnp.float32),
        mesh=_sc_mesh(),
        scratch_types=[
            pltpu.VMEM_SHARED((acc_rows, w), jnp.float32),
            pltpu.VMEM((nidx, 2, _CH), jnp.int32),
            pltpu.VMEM((nbuf, _CH, w), jnp.float32),
            pltpu.SemaphoreType.DMA((nidx,)),
            pltpu.SemaphoreType.DMA((nbuf,)),
            pltpu.SemaphoreType.DMA((nbuf,)),
        ],
        compiler_params=pltpu.CompilerParams(
            needs_layout_passes=False,
            use_tc_tiling_on_sc=None if w % 128 == 0 else False,
        ),
        interpret=interpret,
    )
    def edge_kernel(g_hbm, pack_hbm, out_hbm, acc_sh, idx_v, rows_v,
                    sem_i, sem_g, sem_s):
        cid = lax.axis_index("c")
        sid = lax.axis_index("s")
        base_ch = sid * nch_pair + jnp.where(cid == 0, 0, nch0)
        nch = jnp.where(cid == 0, nch0, nch1)

        def zb(i, _):
            r = i // (w // 16)
            col = (i % (w // 16)) * 16
            rows_v[0, r, pl.ds(col, 16)] = jnp.zeros((16,), jnp.float32)
            return 0

        lax.fori_loop(0, _CH * (w // 16), zb, 0)

        def zacc(i, _):
            pltpu.sync_copy(
                rows_v.at[0, pl.ds(0, _CH)],
                acc_sh.at[pl.ds(sid * rpt + i * _CH, _CH)],
            )
            return 0

        lax.fori_loop(0, rpt // _CH, zacc, 0)

        def idx_desc(c):
            q = lax.rem(c, nidx)
            return pltpu.make_async_copy(
                pack_hbm.at[base_ch + c], idx_v.at[q], sem_i.at[q]
            )

        def gather_desc(c):
            q = lax.rem(c, nidx)
            b = lax.rem(c, nbuf)
            return pltpu.make_async_copy(
                g_hbm.at[idx_v.at[q, 0]], rows_v.at[b], sem_g.at[b]
            )

        def scat_desc(c):
            q = lax.rem(c, nidx)
            b = lax.rem(c, nbuf)
            return pltpu.make_async_copy(
                rows_v.at[b], acc_sh.at[idx_v.at[q, 1]], sem_s.at[b]
            )

        def scat_start(c):
            q = lax.rem(c, nidx)
            b = lax.rem(c, nbuf)
            pltpu.async_copy(
                rows_v.at[b], acc_sh.at[idx_v.at[q, 1]], sem_s.at[b], add=True
            )

        for c in range(k + 2):
            idx_desc(c).start()
        for c in range(k):
            idx_desc(c).wait()
            gather_desc(c).start()

        plsc.subcore_barrier()

        def body(c, _):
            gather_desc(c).wait()
            scat_start(c)

            @pl.when(c + k < nch)
            def _():
                idx_desc(c + k).wait()

                @pl.when(c + k + 2 < nch)
                def _():
                    idx_desc(c + k + 2).start()

                @pl.when(c + k >= nbuf)
                def _():
                    scat_desc(c + k - nbuf).wait()

                gather_desc(c + k).start()

            return 0

        lax.fori_loop(0, nch, body, 0)

        def drain(t, _):
            scat_desc(nch - nbuf + t).wait()
            return 0

        lax.fori_loop(0, nbuf, drain, 0)

        plsc.subcore_barrier()

        def cout(i, _):
            r = sid * rpt + i * _CH
            pltpu.sync_copy(acc_sh.at[pl.ds(r, _CH)],
                            rows_v.at[0, pl.ds(0, _CH)])
            pltpu.sync_copy(rows_v.at[0, pl.ds(0, _CH)],
                            out_hbm.at[cid, pl.ds(r, _CH)])
            return 0

        lax.fori_loop(0, rpt // _CH, cout, 0)

    return edge_kernel


def _scale_matmul_kernel(x_ref, degp_ref, w_ref, out_ref, dis_ref):
    dis = lax.rsqrt(jnp.sum(degp_ref[...], axis=0) + 1.0)
    prod = jnp.dot(x_ref[...], w_ref[...], preferred_element_type=jnp.float32,
                   precision=lax.Precision.HIGHEST)
    dis_ref[...] = dis
    out_ref[...] = dis * prod


def _layer1_combine_kernel(p_ref, g_ref, dis_ref, b_ref, w_ref, out_ref):
    t = p_ref[0] + p_ref[1] + g_ref[...]
    h = jnp.maximum(dis_ref[...] * t + b_ref[...], 0.0)
    prod = jnp.dot(h, w_ref[...], preferred_element_type=jnp.float32,
                   precision=lax.Precision.HIGHEST)
    out_ref[...] = dis_ref[...] * prod


def _layer2_combine_kernel(p_ref, g_ref, dis_ref, b_ref, out_ref):
    t = p_ref[0] + p_ref[1] + g_ref[...]
    out_ref[...] = dis_ref[...] * t + b_ref[...]


def kernel(x, edge_index, W1, b1, W2, b2):
    n, nfeat = x.shape
    nhid = W1.shape[1]
    nclass = W2.shape[1]
    e = edge_index.shape[1]

    src = edge_index[0].astype(jnp.int32)
    dst = edge_index[1].astype(jnp.int32)

    ept = -(-e // (_NW * _CH)) * _CH
    e_pad = ept * _NW
    pad = e_pad - e
    src_p = jnp.concatenate([src, jnp.zeros((pad,), jnp.int32)])
    dst_p = jnp.concatenate([dst, jnp.full((pad,), n, jnp.int32)])
    pack = jnp.stack(
        [src_p.reshape(e_pad // _CH, _CH), dst_p.reshape(e_pad // _CH, _CH)],
        axis=1,
    )
    nch_pair = 2 * (ept // _CH)

    acc_rows = -(-(n + 1) // (_NS * 64)) * (_NS * 64)
    n_hist = -(-(n + 1) // 16) * 16

    deg_parts = _make_degree_kernel(ept, n_hist)(dst_p)
    degp3 = deg_parts[:, :n].reshape(_NW, n, 1)

    blk = 1000
    grid = (n // blk,)

    def rowspec(width):
        return pl.BlockSpec((blk, width), lambda i: (i, 0))

    dis_spec = pl.BlockSpec((blk, 1), lambda i: (i, 0))

    def fullspec(r, c):
        return pl.BlockSpec((r, c), lambda i: (0, 0))

    degp_spec = pl.BlockSpec((_NW, blk, 1), lambda i: (0, i, 0))
    g1, dis2 = pl.pallas_call(
        _scale_matmul_kernel,
        grid=grid,
        in_specs=[rowspec(nfeat), degp_spec, fullspec(nfeat, nhid)],
        out_specs=[rowspec(nhid), dis_spec],
        out_shape=[jax.ShapeDtypeStruct((n, nhid), jnp.float32),
                   jax.ShapeDtypeStruct((n, 1), jnp.float32)],
    )(x, degp3, W1)

    p1 = _make_edge_kernel(nch_pair, nhid, acc_rows, nbuf=5, frac0=_F0)(g1, pack)

    p1_spec = pl.BlockSpec((_NC, blk, nhid), lambda i: (0, i, 0))
    g2 = pl.pallas_call(
        _layer1_combine_kernel,
        grid=grid,
        in_specs=[p1_spec, rowspec(nhid), dis_spec,
                  pl.BlockSpec((1, nhid), lambda i: (0, 0)),
                  fullspec(nhid, nclass)],
        out_specs=rowspec(nclass),
        out_shape=jax.ShapeDtypeStruct((n, nclass), jnp.float32),
    )(p1, g1, dis2, b1.reshape(1, nhid), W2)

    p2 = _make_edge_kernel(nch_pair, nclass, acc_rows, nbuf=8, frac0=_F0)(g2, pack)

    p2_spec = pl.BlockSpec((_NC, blk, nclass), lambda i: (0, i, 0))
    out = pl.pallas_call(
        _layer2_combine_kernel,
        grid=grid,
        in_specs=[p2_spec, rowspec(nclass), dis_spec,
                  pl.BlockSpec((1, nclass), lambda i: (0, 0))],
        out_specs=rowspec(nclass),
        out_shape=jax.ShapeDtypeStruct((n, nclass), jnp.float32),
    )(p2, g2, dis2, b2.reshape(1, nclass))

    return out


# speedup vs baseline: 1.3292x; 1.3292x over previous
import functools

import jax
import jax.numpy as jnp
from jax import lax
from jax.experimental import pallas as pl
from jax.experimental.pallas import tpu as pltpu
from jax.experimental.pallas import tpu_sc as plsc

_NC = 2
_NS = 16
_NW = _NC * _NS
_CH = 64
_F0 = 0.70


def _sc_mesh():
    return plsc.VectorSubcoreMesh(
        core_axis_name="c", subcore_axis_name="s", num_cores=_NC, num_subcores=_NS
    )


def _make_degree_kernel(ept, n_pad, interpret=False):
    @functools.partial(
        pl.kernel,
        out_type=jax.ShapeDtypeStruct((_NW, n_pad), jnp.float32),
        mesh=_sc_mesh(),
        scratch_types=[
            pltpu.VMEM((n_pad,), jnp.float32),
            pltpu.VMEM((ept,), jnp.int32),
        ],
        compiler_params=pltpu.CompilerParams(needs_layout_passes=False),
        interpret=interpret,
    )
    def deg_kernel(dst_hbm, out_hbm, deg_v, idx_v):
        wid = lax.axis_index("s") * _NC + lax.axis_index("c")

        def zero_body(i, _):
            deg_v[pl.ds(i * 16, 16)] = jnp.zeros((16,), jnp.float32)
            return 0

        lax.fori_loop(0, n_pad // 16, zero_body, 0)

        pltpu.sync_copy(dst_hbm.at[pl.ds(wid * ept, ept)], idx_v)

        ones = jnp.ones((16,), jnp.float32)

        def count_body(i, _):
            idx16 = idx_v[pl.ds(i * 16, 16)]
            plsc.addupdate_scatter(deg_v, [idx16], ones)
            return 0

        lax.fori_loop(0, ept // 16, count_body, 0)

        pltpu.sync_copy(deg_v, out_hbm.at[wid])

    return deg_kernel


def _make_edge_kernel(nch_pair, w, acc_rows, nbuf, frac0, interpret=False):
    nch0 = int(round(nch_pair * frac0))
    nch1 = nch_pair - nch0
    rpt = acc_rows // _NS
    k = nbuf - 1
    nidx = nbuf + 3

    @functools.partial(
        pl.kernel,
        out_type=jax.ShapeDtypeStruct((_NC, acc_rows, w), jnp.float32),
        mesh=_sc_mesh(),
        scratch_types=[
            pltpu.VMEM_SHARED((acc_rows, w), jnp.float32),
            pltpu.VMEM((nidx, 2, _CH), jnp.int32),
            pltpu.VMEM((nbuf, _CH, w), jnp.float32),
            pltpu.SemaphoreType.DMA((nidx,)),
            pltpu.SemaphoreType.DMA((nbuf,)),
            pltpu.SemaphoreType.DMA((nbuf,)),
        ],
        compiler_params=pltpu.CompilerParams(
            needs_layout_passes=False,
            use_tc_tiling_on_sc=None if w % 128 == 0 else False,
        ),
        interpret=interpret,
    )
    def edge_kernel(g_hbm, pack_hbm, out_hbm, acc_sh, idx_v, rows_v,
                    sem_i, sem_g, sem_s):
        cid = lax.axis_index("c")
        sid = lax.axis_index("s")
        base_ch = sid * nch_pair + jnp.where(cid == 0, 0, nch0)
        nch = jnp.where(cid == 0, nch0, nch1)

        def zb(i, _):
            r = i // (w // 16)
            col = (i % (w // 16)) * 16
            rows_v[0, r, pl.ds(col, 16)] = jnp.zeros((16,), jnp.float32)
            return 0

        lax.fori_loop(0, _CH * (w // 16), zb, 0)

        def zacc(i, _):
            pltpu.sync_copy(
                rows_v.at[0, pl.ds(0, _CH)],
                acc_sh.at[pl.ds(sid * rpt + i * _CH, _CH)],
            )
            return 0

        lax.fori_loop(0, rpt // _CH, zacc, 0)

        def idx_desc(c):
            q = lax.rem(c, nidx)
            return pltpu.make_async_copy(
                pack_hbm.at[base_ch + c], idx_v.at[q], sem_i.at[q]
            )

        def gather_desc(c):
            q = lax.rem(c, nidx)
            b = lax.rem(c, nbuf)
            return pltpu.make_async_copy(
                g_hbm.at[idx_v.at[q, 0]], rows_v.at[b], sem_g.at[b]
            )

        def scat_desc(c):
            q = lax.rem(c, nidx)
            b = lax.rem(c, nbuf)
            return pltpu.make_async_copy(
                rows_v.at[b], acc_sh.at[idx_v.at[q, 1]], sem_s.at[b]
            )

        def scat_start(c):
            q = lax.rem(c, nidx)
            b = lax.rem(c, nbuf)
            pltpu.async_copy(
                rows_v.at[b], acc_sh.at[idx_v.at[q, 1]], sem_s.at[b], add=True
            )

        for c in range(k + 2):
            idx_desc(c).start()
        for c in range(k):
            idx_desc(c).wait()
            gather_desc(c).start()

        plsc.subcore_barrier()

        def body(c, _):
            gather_desc(c).wait()
            scat_start(c)

            @pl.when(c + k < nch)
            def _():
                idx_desc(c + k).wait()

                @pl.when(c + k + 2 < nch)
                def _():
                    idx_desc(c + k + 2).start()

                @pl.when(c + k >= nbuf)
                def _():
                    scat_desc(c + k - nbuf).wait()

                gather_desc(c + k).start()

            return 0

        lax.fori_loop(0, nch, body, 0)

        def drain(t, _):
            scat_desc(nch - nbuf + t).wait()
            return 0

        lax.fori_loop(0, nbuf, drain, 0)

        plsc.subcore_barrier()

        def cout(i, _):
            r = sid * rpt + i * _CH
            pltpu.sync_copy(acc_sh.at[pl.ds(r, _CH)],
                            rows_v.at[0, pl.ds(0, _CH)])
            pltpu.sync_copy(rows_v.at[0, pl.ds(0, _CH)],
                            out_hbm.at[cid, pl.ds(r, _CH)])
            return 0

        lax.fori_loop(0, rpt // _CH, cout, 0)

    return edge_kernel


def _dis_kernel(degp_ref, out_ref, *, n):
    deg = jnp.sum(degp_ref[...], axis=0)[:n] + 1.0
    out_ref[...] = lax.rsqrt(deg)


def _scale_matmul_kernel(x_ref, dis_ref, w_ref, out_ref):
    prod = jnp.dot(x_ref[...], w_ref[...], preferred_element_type=jnp.float32,
                   precision=lax.Precision.HIGHEST)
    out_ref[...] = dis_ref[...] * prod


def _layer1_combine_kernel(p_ref, g_ref, dis_ref, b_ref, w_ref, out_ref):
    t = p_ref[0] + p_ref[1] + g_ref[...]
    h = jnp.maximum(dis_ref[...] * t + b_ref[...], 0.0)
    prod = jnp.dot(h, w_ref[...], preferred_element_type=jnp.float32,
                   precision=lax.Precision.HIGHEST)
    out_ref[...] = dis_ref[...] * prod


def _layer2_combine_kernel(p_ref, g_ref, dis_ref, b_ref, out_ref):
    t = p_ref[0] + p_ref[1] + g_ref[...]
    out_ref[...] = dis_ref[...] * t + b_ref[...]


def kernel(x, edge_index, W1, b1, W2, b2):
    n, nfeat = x.shape
    nhid = W1.shape[1]
    nclass = W2.shape[1]
    e = edge_index.shape[1]

    src = edge_index[0].astype(jnp.int32)
    dst = edge_index[1].astype(jnp.int32)

    ept = -(-e // (_NW * _CH)) * _CH
    e_pad = ept * _NW
    pad = e_pad - e
    src_p = jnp.concatenate([src, jnp.zeros((pad,), jnp.int32)])
    dst_p = jnp.concatenate([dst, jnp.full((pad,), n, jnp.int32)])
    pack = jnp.stack(
        [src_p.reshape(e_pad // _CH, _CH), dst_p.reshape(e_pad // _CH, _CH)],
        axis=1,
    )
    nch_pair = 2 * (ept // _CH)

    acc_rows = -(-(n + 1) // (_NS * 64)) * (_NS * 64)
    n_hist = -(-(n + 1) // 16) * 16

    deg_parts = _make_degree_kernel(ept, n_hist)(dst_p)
    dis = pl.pallas_call(
        functools.partial(_dis_kernel, n=n),
        out_shape=jax.ShapeDtypeStruct((n,), jnp.float32),
    )(deg_parts)
    dis2 = dis.reshape(n, 1)

    blk = 1000
    grid = (n // blk,)

    def rowspec(width):
        return pl.BlockSpec((blk, width), lambda i: (i, 0))

    dis_spec = pl.BlockSpec((blk, 1), lambda i: (i, 0))

    def fullspec(r, c):
        return pl.BlockSpec((r, c), lambda i: (0, 0))

    g1 = pl.pallas_call(
        _scale_matmul_kernel,
        grid=grid,
        in_specs=[rowspec(nfeat), dis_spec, fullspec(nfeat, nhid)],
        out_specs=rowspec(nhid),
        out_shape=jax.ShapeDtypeStruct((n, nhid), jnp.float32),
    )(x, dis2, W1)

    p1 = _make_edge_kernel(nch_pair, nhid, acc_rows, nbuf=5, frac0=_F0)(g1, pack)

    p1_spec = pl.BlockSpec((_NC, blk, nhid), lambda i: (0, i, 0))
    g2 = pl.pallas_call(
        _layer1_combine_kernel,
        grid=grid,
        in_specs=[p1_spec, rowspec(nhid), dis_spec,
                  pl.BlockSpec((1, nhid), lambda i: (0, 0)),
                  fullspec(nhid, nclass)],
        out_specs=rowspec(nclass),
        out_shape=jax.ShapeDtypeStruct((n, nclass), jnp.float32),
    )(p1, g1, dis2, b1.reshape(1, nhid), W2)

    p2 = _make_edge_kernel(nch_pair, nclass, acc_rows, nbuf=8, frac0=_F0)(g2, pack)

    p2_spec = pl.BlockSpec((_NC, blk, nclass), lambda i: (0, i, 0))
    out = pl.pallas_call(
        _layer2_combine_kernel,
        grid=grid,
        in_specs=[p2_spec, rowspec(nclass), dis_spec,
                  pl.BlockSpec((1, nclass), lambda i: (0, 0))],
        out_specs=rowspec(nclass),
        out_shape=jax.ShapeDtypeStruct((n, nclass), jnp.float32),
    )(p2, g2, dis2, b2.reshape(1, nclass))

    return out


# speedup vs baseline: 1.3374x; 1.0061x over previous
import functools

import jax
import jax.numpy as jnp
from jax import lax
from jax.experimental import pallas as pl
from jax.experimental.pallas import tpu as pltpu
from jax.experimental.pallas import tpu_sc as plsc

_NC = 2
_NS = 16
_NW = _NC * _NS
_CH = 64
_F0 = 0.70


def _sc_mesh():
    return plsc.VectorSubcoreMesh(
        core_axis_name="c", subcore_axis_name="s", num_cores=_NC, num_subcores=_NS
    )


def _make_degree_kernel(ept, n_pad, interpret=False):
    @functools.partial(
        pl.kernel,
        out_type=jax.ShapeDtypeStruct((_NW, n_pad), jnp.float32),
        mesh=_sc_mesh(),
        scratch_types=[
            pltpu.VMEM((n_pad,), jnp.float32),
            pltpu.VMEM((ept,), jnp.int32),
        ],
        compiler_params=pltpu.CompilerParams(needs_layout_passes=False),
        interpret=interpret,
    )
    def deg_kernel(dst_hbm, out_hbm, deg_v, idx_v):
        wid = lax.axis_index("s") * _NC + lax.axis_index("c")

        def zero_body(i, _):
            deg_v[pl.ds(i * 16, 16)] = jnp.zeros((16,), jnp.float32)
            return 0

        lax.fori_loop(0, n_pad // 16, zero_body, 0)

        pltpu.sync_copy(dst_hbm.at[pl.ds(wid * ept, ept)], idx_v)

        ones = jnp.ones((16,), jnp.float32)

        def count_body(i, _):
            idx16 = idx_v[pl.ds(i * 16, 16)]
            plsc.addupdate_scatter(deg_v, [idx16], ones)
            return 0

        lax.fori_loop(0, ept // 16, count_body, 0)

        pltpu.sync_copy(deg_v, out_hbm.at[wid])

    return deg_kernel


def _make_edge_kernel(nch_pair, w, acc_rows, nbuf, frac0, interpret=False):
    nch0 = int(round(nch_pair * frac0))
    nch1 = nch_pair - nch0
    rpt = acc_rows // _NS
    k = nbuf - 1
    nidx = nbuf + 3

    @functools.partial(
        pl.kernel,
        out_type=jax.ShapeDtypeStruct((_NC, acc_rows, w), jnp.float32),
        mesh=_sc_mesh(),
        scratch_types=[
            pltpu.VMEM_SHARED((acc_rows, w), jnp.float32),
            pltpu.VMEM((nidx, 2, _CH), jnp.int32),
            pltpu.VMEM((nbuf, _CH, w), jnp.float32),
            pltpu.SemaphoreType.DMA((nidx,)),
            pltpu.SemaphoreType.DMA((nbuf,)),
            pltpu.SemaphoreType.DMA((nbuf,)),
        ],
        compiler_params=pltpu.CompilerParams(
            needs_layout_passes=False,
            use_tc_tiling_on_sc=None if w % 128 == 0 else False,
        ),
        interpret=interpret,
    )
    def edge_kernel(g_hbm, pack_hbm, out_hbm, acc_sh, idx_v, rows_v,
                    sem_i, sem_g, sem_s):
        cid = lax.axis_index("c")
        sid = lax.axis_index("s")
        base_ch = sid * nch_pair + jnp.where(cid == 0, 0, nch0)
        nch = jnp.where(cid == 0, nch0, nch1)

        def zb(i, _):
            r = i // (w // 16)
            col = (i % (w // 16)) * 16
            rows_v[0, r, pl.ds(col, 16)] = jnp.zeros((16,), jnp.float32)
            return 0

        lax.fori_loop(0, _CH * (w // 16), zb, 0)

        def zacc(i, _):
            pltpu.sync_copy(
                rows_v.at[0, pl.ds(0, _CH)],
                acc_sh.at[pl.ds(sid * rpt + i * _CH, _CH)],
            )
            return 0

        lax.fori_loop(0, rpt // _CH, zacc, 0)

        def idx_desc(c):
            q = lax.rem(c, nidx)
            return pltpu.make_async_copy(
                pack_hbm.at[base_ch + c], idx_v.at[q], sem_i.at[q]
            )

        def gather_desc(c):
            q = lax.rem(c, nidx)
            b = lax.rem(c, nbuf)
            return pltpu.make_async_copy(
                g_hbm.at[idx_v.at[q, 0]], rows_v.at[b], sem_g.at[b]
            )

        def scat_desc(c):
            q = lax.rem(c, nidx)
            b = lax.rem(c, nbuf)
            return pltpu.make_async_copy(
                rows_v.at[b], acc_sh.at[idx_v.at[q, 1]], sem_s.at[b]
            )

        def scat_start(c):
            q = lax.rem(c, nidx)
            b = lax.rem(c, nbuf)
            pltpu.async_copy(
                rows_v.at[b], acc_sh.at[idx_v.at[q, 1]], sem_s.at[b], add=True
            )

        for c in range(k + 2):
            idx_desc(c).start()
        for c in range(k):
            idx_desc(c).wait()
            gather_desc(c).start()

        plsc.subcore_barrier()

        def body(c, _):
            gather_desc(c).wait()
            scat_start(c)

            @pl.when(c + k < nch)
            def _():
                idx_desc(c + k).wait()

                @pl.when(c + k + 2 < nch)
                def _():
                    idx_desc(c + k + 2).start()

                @pl.when(c + k >= nbuf)
                def _():
                    scat_desc(c + k - nbuf).wait()

                gather_desc(c + k).start()

            return 0

        lax.fori_loop(0, nch, body, 0)

        def drain(t, _):
            scat_desc(nch - nbuf + t).wait()
            return 0

        lax.fori_loop(0, nbuf, drain, 0)

        plsc.subcore_barrier()

        def cout(i, _):
            r = sid * rpt + i * _CH
            pltpu.sync_copy(acc_sh.at[pl.ds(r, _CH)],
                            rows_v.at[0, pl.ds(0, _CH)])
            pltpu.sync_copy(rows_v.at[0, pl.ds(0, _CH)],
                            out_hbm.at[cid, pl.ds(r, _CH)])
            return 0

        lax.fori_loop(0, rpt // _CH, cout, 0)

    return edge_kernel


def _dis_kernel(degp_ref, out_ref, *, n):
    deg = jnp.sum(degp_ref[...], axis=0)[:n] + 1.0
    out_ref[...] = lax.rsqrt(deg)


def _scale_matmul_kernel(x_ref, dis_ref, w_ref, out_ref):
    prod = jnp.dot(x_ref[...], w_ref[...], preferred_element_type=jnp.float32,
                   precision=lax.Precision.HIGHEST)
    out_ref[...] = dis_ref[...] * prod


def _layer1_combine_kernel(p_ref, g_ref, dis_ref, b_ref, w_ref, out_ref):
    t = p_ref[0] + p_ref[1] + g_ref[...]
    h = jnp.maximum(dis_ref[...] * t + b_ref[...], 0.0)
    prod = jnp.dot(h, w_ref[...], preferred_element_type=jnp.float32,
                   precision=lax.Precision.HIGHEST)
    out_ref[...] = dis_ref[...] * prod


def _layer2_combine_kernel(p_ref, g_ref, dis_ref, b_ref, out_ref):
    t = p_ref[0] + p_ref[1] + g_ref[...]
    out_ref[...] = dis_ref[...] * t + b_ref[...]


def kernel(x, edge_index, W1, b1, W2, b2):
    n, nfeat = x.shape
    nhid = W1.shape[1]
    nclass = W2.shape[1]
    e = edge_index.shape[1]

    src = edge_index[0].astype(jnp.int32)
    dst = edge_index[1].astype(jnp.int32)

    ept = -(-e // (_NW * _CH)) * _CH
    e_pad = ept * _NW
    pad = e_pad - e
    src_p = jnp.concatenate([src, jnp.zeros((pad,), jnp.int32)])
    dst_p = jnp.concatenate([dst, jnp.full((pad,), n, jnp.int32)])
    pack = jnp.stack(
        [src_p.reshape(e_pad // _CH, _CH), dst_p.reshape(e_pad // _CH, _CH)],
        axis=1,
    )
    nch_pair = 2 * (ept // _CH)

    acc_rows = -(-(n + 1) // (_NS * 64)) * (_NS * 64)
    n_hist = -(-(n + 1) // 16) * 16

    deg_parts = _make_degree_kernel(ept, n_hist)(dst_p)
    dis = pl.pallas_call(
        functools.partial(_dis_kernel, n=n),
        out_shape=jax.ShapeDtypeStruct((n,), jnp.float32),
    )(deg_parts)
    dis2 = dis.reshape(n, 1)

    blk = 1000
    grid = (n // blk,)

    def rowspec(width):
        return pl.BlockSpec((blk, width), lambda i: (i, 0))

    dis_spec = pl.BlockSpec((blk, 1), lambda i: (i, 0))

    def fullspec(r, c):
        return pl.BlockSpec((r, c), lambda i: (0, 0))

    g1 = pl.pallas_call(
        _scale_matmul_kernel,
        grid=grid,
        in_specs=[rowspec(nfeat), dis_spec, fullspec(nfeat, nhid)],
        out_specs=rowspec(nhid),
        out_shape=jax.ShapeDtypeStruct((n, nhid), jnp.float32),
    )(x, dis2, W1)

    p1 = _make_edge_kernel(nch_pair, nhid, acc_rows, nbuf=5, frac0=_F0)(g1, pack)

    p1_spec = pl.BlockSpec((_NC, blk, nhid), lambda i: (0, i, 0))
    g2 = pl.pallas_call(
        _layer1_combine_kernel,
        grid=grid,
        in_specs=[p1_spec, rowspec(nhid), dis_spec,
                  pl.BlockSpec((1, nhid), lambda i: (0, 0)),
                  fullspec(nhid, nclass)],
        out_specs=rowspec(nclass),
        out_shape=jax.ShapeDtypeStruct((n, nclass), jnp.float32),
    )(p1, g1, dis2, b1.reshape(1, nhid), W2)

    p2 = _make_edge_kernel(nch_pair, nclass, acc_rows, nbuf=8, frac0=0.73)(g2, pack)

    p2_spec = pl.BlockSpec((_NC, blk, nclass), lambda i: (0, i, 0))
    out = pl.pallas_call(
        _layer2_combine_kernel,
        grid=grid,
        in_specs=[p2_spec, rowspec(nclass), dis_spec,
                  pl.BlockSpec((1, nclass), lambda i: (0, 0))],
        out_specs=rowspec(nclass),
        out_shape=jax.ShapeDtypeStruct((n, nclass), jnp.float32),
    )(p2, g2, dis2, b2.reshape(1, nclass))

    return out


# speedup vs baseline: 1.3466x; 1.0069x over previous
import functools

import jax
import jax.numpy as jnp
from jax import lax
from jax.experimental import pallas as pl
from jax.experimental.pallas import tpu as pltpu
from jax.experimental.pallas import tpu_sc as plsc

_NC = 2
_NS = 16
_NW = _NC * _NS
_CH = 64
_F0 = 0.72


def _sc_mesh():
    return plsc.VectorSubcoreMesh(
        core_axis_name="c", subcore_axis_name="s", num_cores=_NC, num_subcores=_NS
    )


def _make_degree_kernel(ept, n_pad, interpret=False):
    @functools.partial(
        pl.kernel,
        out_type=jax.ShapeDtypeStruct((_NW, n_pad), jnp.float32),
        mesh=_sc_mesh(),
        scratch_types=[
            pltpu.VMEM((n_pad,), jnp.float32),
            pltpu.VMEM((ept,), jnp.int32),
        ],
        compiler_params=pltpu.CompilerParams(needs_layout_passes=False),
        interpret=interpret,
    )
    def deg_kernel(dst_hbm, out_hbm, deg_v, idx_v):
        wid = lax.axis_index("s") * _NC + lax.axis_index("c")

        def zero_body(i, _):
            deg_v[pl.ds(i * 16, 16)] = jnp.zeros((16,), jnp.float32)
            return 0

        lax.fori_loop(0, n_pad // 16, zero_body, 0)

        pltpu.sync_copy(dst_hbm.at[pl.ds(wid * ept, ept)], idx_v)

        ones = jnp.ones((16,), jnp.float32)

        def count_body(i, _):
            idx16 = idx_v[pl.ds(i * 16, 16)]
            plsc.addupdate_scatter(deg_v, [idx16], ones)
            return 0

        lax.fori_loop(0, ept // 16, count_body, 0)

        pltpu.sync_copy(deg_v, out_hbm.at[wid])

    return deg_kernel


def _make_edge_kernel(nch_pair, w, acc_rows, nbuf, frac0, interpret=False):
    nch0 = int(round(nch_pair * frac0))
    nch1 = nch_pair - nch0
    rpt = acc_rows // _NS
    k = nbuf - 1
    nidx = nbuf + 3

    @functools.partial(
        pl.kernel,
        out_type=jax.ShapeDtypeStruct((_NC, acc_rows, w), jnp.float32),
        mesh=_sc_mesh(),
        scratch_types=[
            pltpu.VMEM_SHARED((acc_rows, w), jnp.float32),
            pltpu.VMEM((nidx, 2, _CH), jnp.int32),
            pltpu.VMEM((nbuf, _CH, w), jnp.float32),
            pltpu.SemaphoreType.DMA((nidx,)),
            pltpu.SemaphoreType.DMA((nbuf,)),
            pltpu.SemaphoreType.DMA((nbuf,)),
        ],
        compiler_params=pltpu.CompilerParams(
            needs_layout_passes=False,
            use_tc_tiling_on_sc=None if w % 128 == 0 else False,
        ),
        interpret=interpret,
    )
    def edge_kernel(g_hbm, pack_hbm, out_hbm, acc_sh, idx_v, rows_v,
                    sem_i, sem_g, sem_s):
        cid = lax.axis_index("c")
        sid = lax.axis_index("s")
        base_ch = sid * nch_pair + jnp.where(cid == 0, 0, nch0)
        nch = jnp.where(cid == 0, nch0, nch1)

        def zb(i, _):
            r = i // (w // 16)
            col = (i % (w // 16)) * 16
            rows_v[0, r, pl.ds(col, 16)] = jnp.zeros((16,), jnp.float32)
            return 0

        lax.fori_loop(0, _CH * (w // 16), zb, 0)

        def zacc(i, _):
            pltpu.sync_copy(
                rows_v.at[0, pl.ds(0, _CH)],
                acc_sh.at[pl.ds(sid * rpt + i * _CH, _CH)],
            )
            return 0

        lax.fori_loop(0, rpt // _CH, zacc, 0)

        def idx_desc(c):
            q = lax.rem(c, nidx)
            return pltpu.make_async_copy(
                pack_hbm.at[base_ch + c], idx_v.at[q], sem_i.at[q]
            )

        def gather_desc(c):
            q = lax.rem(c, nidx)
            b = lax.rem(c, nbuf)
            return pltpu.make_async_copy(
                g_hbm.at[idx_v.at[q, 0]], rows_v.at[b], sem_g.at[b]
            )

        def scat_desc(c):
            q = lax.rem(c, nidx)
            b = lax.rem(c, nbuf)
            return pltpu.make_async_copy(
                rows_v.at[b], acc_sh.at[idx_v.at[q, 1]], sem_s.at[b]
            )

        def scat_start(c):
            q = lax.rem(c, nidx)
            b = lax.rem(c, nbuf)
            pltpu.async_copy(
                rows_v.at[b], acc_sh.at[idx_v.at[q, 1]], sem_s.at[b], add=True
            )

        for c in range(k + 2):
            idx_desc(c).start()
        for c in range(k):
            idx_desc(c).wait()
            gather_desc(c).start()

        plsc.subcore_barrier()

        def body(c, _):
            gather_desc(c).wait()
            scat_start(c)

            @pl.when(c + k < nch)
            def _():
                idx_desc(c + k).wait()

                @pl.when(c + k + 2 < nch)
                def _():
                    idx_desc(c + k + 2).start()

                @pl.when(c + k >= nbuf)
                def _():
                    scat_desc(c + k - nbuf).wait()

                gather_desc(c + k).start()

            return 0

        lax.fori_loop(0, nch, body, 0)

        def drain(t, _):
            scat_desc(nch - nbuf + t).wait()
            return 0

        lax.fori_loop(0, nbuf, drain, 0)

        plsc.subcore_barrier()

        def cout(i, _):
            r = sid * rpt + i * _CH
            pltpu.sync_copy(acc_sh.at[pl.ds(r, _CH)],
                            rows_v.at[0, pl.ds(0, _CH)])
            pltpu.sync_copy(rows_v.at[0, pl.ds(0, _CH)],
                            out_hbm.at[cid, pl.ds(r, _CH)])
            return 0

        lax.fori_loop(0, rpt // _CH, cout, 0)

    return edge_kernel


def _dis_kernel(degp_ref, out_ref, *, n):
    deg = jnp.sum(degp_ref[...], axis=0)[:n] + 1.0
    out_ref[...] = lax.rsqrt(deg)


def _scale_matmul_kernel(x_ref, dis_ref, w_ref, out_ref):
    prod = jnp.dot(x_ref[...], w_ref[...], preferred_element_type=jnp.float32,
                   precision=lax.Precision.HIGHEST)
    out_ref[...] = dis_ref[...] * prod


def _layer1_combine_kernel(p_ref, g_ref, dis_ref, b_ref, w_ref, out_ref):
    t = p_ref[0] + p_ref[1] + g_ref[...]
    h = jnp.maximum(dis_ref[...] * t + b_ref[...], 0.0)
    prod = jnp.dot(h, w_ref[...], preferred_element_type=jnp.float32,
                   precision=lax.Precision.HIGHEST)
    out_ref[...] = dis_ref[...] * prod


def _layer2_combine_kernel(p_ref, g_ref, dis_ref, b_ref, out_ref):
    t = p_ref[0] + p_ref[1] + g_ref[...]
    out_ref[...] = dis_ref[...] * t + b_ref[...]


def kernel(x, edge_index, W1, b1, W2, b2):
    n, nfeat = x.shape
    nhid = W1.shape[1]
    nclass = W2.shape[1]
    e = edge_index.shape[1]

    src = edge_index[0].astype(jnp.int32)
    dst = edge_index[1].astype(jnp.int32)

    ept = -(-e // (_NW * _CH)) * _CH
    e_pad = ept * _NW
    pad = e_pad - e
    src_p = jnp.concatenate([src, jnp.zeros((pad,), jnp.int32)])
    dst_p = jnp.concatenate([dst, jnp.full((pad,), n, jnp.int32)])
    pack = jnp.stack(
        [src_p.reshape(e_pad // _CH, _CH), dst_p.reshape(e_pad // _CH, _CH)],
        axis=1,
    )
    nch_pair = 2 * (ept // _CH)

    acc_rows = -(-(n + 1) // (_NS * 64)) * (_NS * 64)
    n_hist = -(-(n + 1) // 16) * 16

    deg_parts = _make_degree_kernel(ept, n_hist)(dst_p)
    dis = pl.pallas_call(
        functools.partial(_dis_kernel, n=n),
        out_shape=jax.ShapeDtypeStruct((n,), jnp.float32),
    )(deg_parts)
    dis2 = dis.reshape(n, 1)

    blk = 1000
    grid = (n // blk,)

    def rowspec(width):
        return pl.BlockSpec((blk, width), lambda i: (i, 0))

    dis_spec = pl.BlockSpec((blk, 1), lambda i: (i, 0))

    def fullspec(r, c):
        return pl.BlockSpec((r, c), lambda i: (0, 0))

    g1 = pl.pallas_call(
        _scale_matmul_kernel,
        grid=grid,
        in_specs=[rowspec(nfeat), dis_spec, fullspec(nfeat, nhid)],
        out_specs=rowspec(nhid),
        out_shape=jax.ShapeDtypeStruct((n, nhid), jnp.float32),
    )(x, dis2, W1)

    p1 = _make_edge_kernel(nch_pair, nhid, acc_rows, nbuf=5, frac0=_F0)(g1, pack)

    p1_spec = pl.BlockSpec((_NC, blk, nhid), lambda i: (0, i, 0))
    g2 = pl.pallas_call(
        _layer1_combine_kernel,
        grid=grid,
        in_specs=[p1_spec, rowspec(nhid), dis_spec,
                  pl.BlockSpec((1, nhid), lambda i: (0, 0)),
                  fullspec(nhid, nclass)],
        out_specs=rowspec(nclass),
        out_shape=jax.ShapeDtypeStruct((n, nclass), jnp.float32),
    )(p1, g1, dis2, b1.reshape(1, nhid), W2)

    p2 = _make_edge_kernel(nch_pair, nclass, acc_rows, nbuf=8, frac0=0.73)(g2, pack)

    p2_spec = pl.BlockSpec((_NC, blk, nclass), lambda i: (0, i, 0))
    out = pl.pallas_call(
        _layer2_combine_kernel,
        grid=grid,
        in_specs=[p2_spec, rowspec(nclass), dis_spec,
                  pl.BlockSpec((1, nclass), lambda i: (0, 0))],
        out_specs=rowspec(nclass),
        out_shape=jax.ShapeDtypeStruct((n, nclass), jnp.float32),
    )(p2, g2, dis2, b2.reshape(1, nclass))

    return out


# speedup vs baseline: 1.3588x; 1.0090x over previous
import functools

import jax
import jax.numpy as jnp
from jax import lax
from jax.experimental import pallas as pl
from jax.experimental.pallas import tpu as pltpu
from jax.experimental.pallas import tpu_sc as plsc

_NC = 2
_NS = 16
_NW = _NC * _NS
_CH = 64
_F0 = 0.74


def _sc_mesh():
    return plsc.VectorSubcoreMesh(
        core_axis_name="c", subcore_axis_name="s", num_cores=_NC, num_subcores=_NS
    )


def _make_degree_kernel(ept, n_pad, interpret=False):
    @functools.partial(
        pl.kernel,
        out_type=jax.ShapeDtypeStruct((_NW, n_pad), jnp.float32),
        mesh=_sc_mesh(),
        scratch_types=[
            pltpu.VMEM((n_pad,), jnp.float32),
            pltpu.VMEM((ept,), jnp.int32),
        ],
        compiler_params=pltpu.CompilerParams(needs_layout_passes=False),
        interpret=interpret,
    )
    def deg_kernel(dst_hbm, out_hbm, deg_v, idx_v):
        wid = lax.axis_index("s") * _NC + lax.axis_index("c")

        def zero_body(i, _):
            deg_v[pl.ds(i * 16, 16)] = jnp.zeros((16,), jnp.float32)
            return 0

        lax.fori_loop(0, n_pad // 16, zero_body, 0)

        pltpu.sync_copy(dst_hbm.at[pl.ds(wid * ept, ept)], idx_v)

        ones = jnp.ones((16,), jnp.float32)

        def count_body(i, _):
            idx16 = idx_v[pl.ds(i * 16, 16)]
            plsc.addupdate_scatter(deg_v, [idx16], ones)
            return 0

        lax.fori_loop(0, ept // 16, count_body, 0)

        pltpu.sync_copy(deg_v, out_hbm.at[wid])

    return deg_kernel


def _make_edge_kernel(nch_pair, w, acc_rows, nbuf, frac0, interpret=False):
    nch0 = int(round(nch_pair * frac0))
    nch1 = nch_pair - nch0
    rpt = acc_rows // _NS
    k = nbuf - 1
    nidx = nbuf + 3

    @functools.partial(
        pl.kernel,
        out_type=jax.ShapeDtypeStruct((_NC, acc_rows, w), jnp.float32),
        mesh=_sc_mesh(),
        scratch_types=[
            pltpu.VMEM_SHARED((acc_rows, w), jnp.float32),
            pltpu.VMEM((nidx, 2, _CH), jnp.int32),
            pltpu.VMEM((nbuf, _CH, w), jnp.float32),
            pltpu.SemaphoreType.DMA((nidx,)),
            pltpu.SemaphoreType.DMA((nbuf,)),
            pltpu.SemaphoreType.DMA((nbuf,)),
        ],
        compiler_params=pltpu.CompilerParams(
            needs_layout_passes=False,
            use_tc_tiling_on_sc=None if w % 128 == 0 else False,
        ),
        interpret=interpret,
    )
    def edge_kernel(g_hbm, pack_hbm, out_hbm, acc_sh, idx_v, rows_v,
                    sem_i, sem_g, sem_s):
        cid = lax.axis_index("c")
        sid = lax.axis_index("s")
        base_ch = sid * nch_pair + jnp.where(cid == 0, 0, nch0)
        nch = jnp.where(cid == 0, nch0, nch1)

        def zb(i, _):
            r = i // (w // 16)
            col = (i % (w // 16)) * 16
            rows_v[0, r, pl.ds(col, 16)] = jnp.zeros((16,), jnp.float32)
            return 0

        lax.fori_loop(0, _CH * (w // 16), zb, 0)

        def zacc(i, _):
            pltpu.sync_copy(
                rows_v.at[0, pl.ds(0, _CH)],
                acc_sh.at[pl.ds(sid * rpt + i * _CH, _CH)],
            )
            return 0

        lax.fori_loop(0, rpt // _CH, zacc, 0)

        def idx_desc(c):
            q = lax.rem(c, nidx)
            return pltpu.make_async_copy(
                pack_hbm.at[base_ch + c], idx_v.at[q], sem_i.at[q]
            )

        def gather_desc(c):
            q = lax.rem(c, nidx)
            b = lax.rem(c, nbuf)
            return pltpu.make_async_copy(
                g_hbm.at[idx_v.at[q, 0]], rows_v.at[b], sem_g.at[b]
            )

        def scat_desc(c):
            q = lax.rem(c, nidx)
            b = lax.rem(c, nbuf)
            return pltpu.make_async_copy(
                rows_v.at[b], acc_sh.at[idx_v.at[q, 1]], sem_s.at[b]
            )

        def scat_start(c):
            q = lax.rem(c, nidx)
            b = lax.rem(c, nbuf)
            pltpu.async_copy(
                rows_v.at[b], acc_sh.at[idx_v.at[q, 1]], sem_s.at[b], add=True
            )

        for c in range(k + 2):
            idx_desc(c).start()
        for c in range(k):
            idx_desc(c).wait()
            gather_desc(c).start()

        plsc.subcore_barrier()

        def body(c, _):
            gather_desc(c).wait()
            scat_start(c)

            @pl.when(c + k < nch)
            def _():
                idx_desc(c + k).wait()

                @pl.when(c + k + 2 < nch)
                def _():
                    idx_desc(c + k + 2).start()

                @pl.when(c + k >= nbuf)
                def _():
                    scat_desc(c + k - nbuf).wait()

                gather_desc(c + k).start()

            return 0

        lax.fori_loop(0, nch, body, 0)

        def drain(t, _):
            scat_desc(nch - nbuf + t).wait()
            return 0

        lax.fori_loop(0, nbuf, drain, 0)

        plsc.subcore_barrier()

        def cout(i, _):
            r = sid * rpt + i * _CH
            pltpu.sync_copy(acc_sh.at[pl.ds(r, _CH)],
                            rows_v.at[0, pl.ds(0, _CH)])
            pltpu.sync_copy(rows_v.at[0, pl.ds(0, _CH)],
                            out_hbm.at[cid, pl.ds(r, _CH)])
            return 0

        lax.fori_loop(0, rpt // _CH, cout, 0)

    return edge_kernel


def _dis_kernel(degp_ref, out_ref, *, n):
    deg = jnp.sum(degp_ref[...], axis=0)[:n] + 1.0
    out_ref[...] = lax.rsqrt(deg)


def _scale_matmul_kernel(x_ref, dis_ref, w_ref, out_ref):
    prod = jnp.dot(x_ref[...], w_ref[...], preferred_element_type=jnp.float32,
                   precision=lax.Precision.HIGHEST)
    out_ref[...] = dis_ref[...] * prod


def _layer1_combine_kernel(p_ref, g_ref, dis_ref, b_ref, w_ref, out_ref):
    t = p_ref[0] + p_ref[1] + g_ref[...]
    h = jnp.maximum(dis_ref[...] * t + b_ref[...], 0.0)
    prod = jnp.dot(h, w_ref[...], preferred_element_type=jnp.float32,
                   precision=lax.Precision.HIGHEST)
    out_ref[...] = dis_ref[...] * prod


def _layer2_combine_kernel(p_ref, g_ref, dis_ref, b_ref, out_ref):
    t = p_ref[0] + p_ref[1] + g_ref[...]
    out_ref[...] = dis_ref[...] * t + b_ref[...]


def kernel(x, edge_index, W1, b1, W2, b2):
    n, nfeat = x.shape
    nhid = W1.shape[1]
    nclass = W2.shape[1]
    e = edge_index.shape[1]

    src = edge_index[0].astype(jnp.int32)
    dst = edge_index[1].astype(jnp.int32)

    ept = -(-e // (_NW * _CH)) * _CH
    e_pad = ept * _NW
    pad = e_pad - e
    src_p = jnp.concatenate([src, jnp.zeros((pad,), jnp.int32)])
    dst_p = jnp.concatenate([dst, jnp.full((pad,), n, jnp.int32)])
    pack = jnp.stack(
        [src_p.reshape(e_pad // _CH, _CH), dst_p.reshape(e_pad // _CH, _CH)],
        axis=1,
    )
    nch_pair = 2 * (ept // _CH)

    acc_rows = -(-(n + 1) // (_NS * 64)) * (_NS * 64)
    n_hist = -(-(n + 1) // 16) * 16

    deg_parts = _make_degree_kernel(ept, n_hist)(dst_p)
    dis = pl.pallas_call(
        functools.partial(_dis_kernel, n=n),
        out_shape=jax.ShapeDtypeStruct((n,), jnp.float32),
    )(deg_parts)
    dis2 = dis.reshape(n, 1)

    blk = 1000
    grid = (n // blk,)

    def rowspec(width):
        return pl.BlockSpec((blk, width), lambda i: (i, 0))

    dis_spec = pl.BlockSpec((blk, 1), lambda i: (i, 0))

    def fullspec(r, c):
        return pl.BlockSpec((r, c), lambda i: (0, 0))

    g1 = pl.pallas_call(
        _scale_matmul_kernel,
        grid=grid,
        in_specs=[rowspec(nfeat), dis_spec, fullspec(nfeat, nhid)],
        out_specs=rowspec(nhid),
        out_shape=jax.ShapeDtypeStruct((n, nhid), jnp.float32),
    )(x, dis2, W1)

    p1 = _make_edge_kernel(nch_pair, nhid, acc_rows, nbuf=5, frac0=_F0)(g1, pack)

    p1_spec = pl.BlockSpec((_NC, blk, nhid), lambda i: (0, i, 0))
    g2 = pl.pallas_call(
        _layer1_combine_kernel,
        grid=grid,
        in_specs=[p1_spec, rowspec(nhid), dis_spec,
                  pl.BlockSpec((1, nhid), lambda i: (0, 0)),
                  fullspec(nhid, nclass)],
        out_specs=rowspec(nclass),
        out_shape=jax.ShapeDtypeStruct((n, nclass), jnp.float32),
    )(p1, g1, dis2, b1.reshape(1, nhid), W2)

    p2 = _make_edge_kernel(nch_pair, nclass, acc_rows, nbuf=8, frac0=0.75)(g2, pack)

    p2_spec = pl.BlockSpec((_NC, blk, nclass), lambda i: (0, i, 0))
    out = pl.pallas_call(
        _layer2_combine_kernel,
        grid=grid,
        in_specs=[p2_spec, rowspec(nclass), dis_spec,
                  pl.BlockSpec((1, nclass), lambda i: (0, 0))],
        out_specs=rowspec(nclass),
        out_shape=jax.ShapeDtypeStruct((n, nclass), jnp.float32),
    )(p2, g2, dis2, b2.reshape(1, nclass))

    return out


# speedup vs baseline: 1.3601x; 1.0010x over previous
import functools

import jax
import jax.numpy as jnp
from jax import lax
from jax.experimental import pallas as pl
from jax.experimental.pallas import tpu as pltpu
from jax.experimental.pallas import tpu_sc as plsc

_NC = 2
_NS = 16
_NW = _NC * _NS
_CH = 64
_F0 = 0.78


def _sc_mesh():
    return plsc.VectorSubcoreMesh(
        core_axis_name="c", subcore_axis_name="s", num_cores=_NC, num_subcores=_NS
    )


def _make_degree_kernel(ept, n_pad, interpret=False):
    @functools.partial(
        pl.kernel,
        out_type=jax.ShapeDtypeStruct((_NW, n_pad), jnp.float32),
        mesh=_sc_mesh(),
        scratch_types=[
            pltpu.VMEM((n_pad,), jnp.float32),
            pltpu.VMEM((ept,), jnp.int32),
        ],
        compiler_params=pltpu.CompilerParams(needs_layout_passes=False),
        interpret=interpret,
    )
    def deg_kernel(dst_hbm, out_hbm, deg_v, idx_v):
        wid = lax.axis_index("s") * _NC + lax.axis_index("c")

        def zero_body(i, _):
            deg_v[pl.ds(i * 16, 16)] = jnp.zeros((16,), jnp.float32)
            return 0

        lax.fori_loop(0, n_pad // 16, zero_body, 0)

        pltpu.sync_copy(dst_hbm.at[pl.ds(wid * ept, ept)], idx_v)

        ones = jnp.ones((16,), jnp.float32)

        def count_body(i, _):
            idx16 = idx_v[pl.ds(i * 16, 16)]
            plsc.addupdate_scatter(deg_v, [idx16], ones)
            return 0

        lax.fori_loop(0, ept // 16, count_body, 0)

        pltpu.sync_copy(deg_v, out_hbm.at[wid])

    return deg_kernel


def _make_edge_kernel(nch_pair, w, acc_rows, nbuf, frac0, interpret=False):
    nch0 = int(round(nch_pair * frac0))
    nch1 = nch_pair - nch0
    rpt = acc_rows // _NS
    k = nbuf - 1
    nidx = nbuf + 3

    @functools.partial(
        pl.kernel,
        out_type=jax.ShapeDtypeStruct((_NC, acc_rows, w), jnp.float32),
        mesh=_sc_mesh(),
        scratch_types=[
            pltpu.VMEM_SHARED((acc_rows, w), jnp.float32),
            pltpu.VMEM((nidx, 2, _CH), jnp.int32),
            pltpu.VMEM((nbuf, _CH, w), jnp.float32),
            pltpu.SemaphoreType.DMA((nidx,)),
            pltpu.SemaphoreType.DMA((nbuf,)),
            pltpu.SemaphoreType.DMA((nbuf,)),
        ],
        compiler_params=pltpu.CompilerParams(
            needs_layout_passes=False,
            use_tc_tiling_on_sc=None if w % 128 == 0 else False,
        ),
        interpret=interpret,
    )
    def edge_kernel(g_hbm, pack_hbm, out_hbm, acc_sh, idx_v, rows_v,
                    sem_i, sem_g, sem_s):
        cid = lax.axis_index("c")
        sid = lax.axis_index("s")
        base_ch = sid * nch_pair + jnp.where(cid == 0, 0, nch0)
        nch = jnp.where(cid == 0, nch0, nch1)

        def zb(i, _):
            r = i // (w // 16)
            col = (i % (w // 16)) * 16
            rows_v[0, r, pl.ds(col, 16)] = jnp.zeros((16,), jnp.float32)
            return 0

        lax.fori_loop(0, _CH * (w // 16), zb, 0)

        def zacc(i, _):
            pltpu.sync_copy(
                rows_v.at[0, pl.ds(0, _CH)],
                acc_sh.at[pl.ds(sid * rpt + i * _CH, _CH)],
            )
            return 0

        lax.fori_loop(0, rpt // _CH, zacc, 0)

        def idx_desc(c):
            q = lax.rem(c, nidx)
            return pltpu.make_async_copy(
                pack_hbm.at[base_ch + c], idx_v.at[q], sem_i.at[q]
            )

        def gather_desc(c):
            q = lax.rem(c, nidx)
            b = lax.rem(c, nbuf)
            return pltpu.make_async_copy(
                g_hbm.at[idx_v.at[q, 0]], rows_v.at[b], sem_g.at[b]
            )

        def scat_desc(c):
            q = lax.rem(c, nidx)
            b = lax.rem(c, nbuf)
            return pltpu.make_async_copy(
                rows_v.at[b], acc_sh.at[idx_v.at[q, 1]], sem_s.at[b]
            )

        def scat_start(c):
            q = lax.rem(c, nidx)
            b = lax.rem(c, nbuf)
            pltpu.async_copy(
                rows_v.at[b], acc_sh.at[idx_v.at[q, 1]], sem_s.at[b], add=True
            )

        for c in range(k + 2):
            idx_desc(c).start()
        for c in range(k):
            idx_desc(c).wait()
            gather_desc(c).start()

        plsc.subcore_barrier()

        def body(c, _):
            gather_desc(c).wait()
            scat_start(c)

            @pl.when(c + k < nch)
            def _():
                idx_desc(c + k).wait()

                @pl.when(c + k + 2 < nch)
                def _():
                    idx_desc(c + k + 2).start()

                @pl.when(c + k >= nbuf)
                def _():
                    scat_desc(c + k - nbuf).wait()

                gather_desc(c + k).start()

            return 0

        lax.fori_loop(0, nch, body, 0)

        def drain(t, _):
            scat_desc(nch - nbuf + t).wait()
            return 0

        lax.fori_loop(0, nbuf, drain, 0)

        plsc.subcore_barrier()

        def cout(i, _):
            r = sid * rpt + i * _CH
            pltpu.sync_copy(acc_sh.at[pl.ds(r, _CH)],
                            rows_v.at[0, pl.ds(0, _CH)])
            pltpu.sync_copy(rows_v.at[0, pl.ds(0, _CH)],
                            out_hbm.at[cid, pl.ds(r, _CH)])
            return 0

        lax.fori_loop(0, rpt // _CH, cout, 0)

    return edge_kernel


def _dis_kernel(degp_ref, out_ref, *, n):
    deg = jnp.sum(degp_ref[...], axis=0)[:n] + 1.0
    out_ref[...] = lax.rsqrt(deg)


def _scale_matmul_kernel(x_ref, dis_ref, w_ref, out_ref):
    prod = jnp.dot(x_ref[...], w_ref[...], preferred_element_type=jnp.float32,
                   precision=lax.Precision.HIGHEST)
    out_ref[...] = dis_ref[...] * prod


def _layer1_combine_kernel(p_ref, g_ref, dis_ref, b_ref, w_ref, out_ref):
    t = p_ref[0] + p_ref[1] + g_ref[...]
    h = jnp.maximum(dis_ref[...] * t + b_ref[...], 0.0)
    prod = jnp.dot(h, w_ref[...], preferred_element_type=jnp.float32,
                   precision=lax.Precision.HIGHEST)
    out_ref[...] = dis_ref[...] * prod


def _layer2_combine_kernel(p_ref, g_ref, dis_ref, b_ref, out_ref):
    t = p_ref[0] + p_ref[1] + g_ref[...]
    out_ref[...] = dis_ref[...] * t + b_ref[...]


def kernel(x, edge_index, W1, b1, W2, b2):
    n, nfeat = x.shape
    nhid = W1.shape[1]
    nclass = W2.shape[1]
    e = edge_index.shape[1]

    src = edge_index[0].astype(jnp.int32)
    dst = edge_index[1].astype(jnp.int32)

    ept = -(-e // (_NW * _CH)) * _CH
    e_pad = ept * _NW
    pad = e_pad - e
    src_p = jnp.concatenate([src, jnp.zeros((pad,), jnp.int32)])
    dst_p = jnp.concatenate([dst, jnp.full((pad,), n, jnp.int32)])
    pack = jnp.stack(
        [src_p.reshape(e_pad // _CH, _CH), dst_p.reshape(e_pad // _CH, _CH)],
        axis=1,
    )
    nch_pair = 2 * (ept // _CH)

    acc_rows = -(-(n + 1) // (_NS * 64)) * (_NS * 64)
    n_hist = -(-(n + 1) // 16) * 16

    deg_parts = _make_degree_kernel(ept, n_hist)(dst_p)
    dis = pl.pallas_call(
        functools.partial(_dis_kernel, n=n),
        out_shape=jax.ShapeDtypeStruct((n,), jnp.float32),
    )(deg_parts)
    dis2 = dis.reshape(n, 1)

    blk = 1000
    grid = (n // blk,)

    def rowspec(width):
        return pl.BlockSpec((blk, width), lambda i: (i, 0))

    dis_spec = pl.BlockSpec((blk, 1), lambda i: (i, 0))

    def fullspec(r, c):
        return pl.BlockSpec((r, c), lambda i: (0, 0))

    g1 = pl.pallas_call(
        _scale_matmul_kernel,
        grid=grid,
        in_specs=[rowspec(nfeat), dis_spec, fullspec(nfeat, nhid)],
        out_specs=rowspec(nhid),
        out_shape=jax.ShapeDtypeStruct((n, nhid), jnp.float32),
    )(x, dis2, W1)

    p1 = _make_edge_kernel(nch_pair, nhid, acc_rows, nbuf=5, frac0=_F0)(g1, pack)

    p1_spec = pl.BlockSpec((_NC, blk, nhid), lambda i: (0, i, 0))
    g2 = pl.pallas_call(
        _layer1_combine_kernel,
        grid=grid,
        in_specs=[p1_spec, rowspec(nhid), dis_spec,
                  pl.BlockSpec((1, nhid), lambda i: (0, 0)),
                  fullspec(nhid, nclass)],
        out_specs=rowspec(nclass),
        out_shape=jax.ShapeDtypeStruct((n, nclass), jnp.float32),
    )(p1, g1, dis2, b1.reshape(1, nhid), W2)

    p2 = _make_edge_kernel(nch_pair, nclass, acc_rows, nbuf=8, frac0=0.78)(g2, pack)

    p2_spec = pl.BlockSpec((_NC, blk, nclass), lambda i: (0, i, 0))
    out = pl.pallas_call(
        _layer2_combine_kernel,
        grid=grid,
        in_specs=[p2_spec, rowspec(nclass), dis_spec,
                  pl.BlockSpec((1, nclass), lambda i: (0, 0))],
        out_specs=rowspec(nclass),
        out_shape=jax.ShapeDtypeStruct((n, nclass), jnp.float32),
    )(p2, g2, dis2, b2.reshape(1, nclass))

    return out
